# baseline (speedup 1.0000x reference)
import functools

import jax
import jax.numpy as jnp
from jax import lax
from jax.experimental import pallas as pl
from jax.experimental.pallas import tpu as pltpu
from jax.experimental.pallas import tpu_sc as plsc

NUM_PARTITIONS = 1000000
EMBED_DIM = 64
BATCH = 16384
LANES = 128

_INFO = plsc.get_sparse_core_info()
NC, NS, L = _INFO.num_cores, _INFO.num_subcores, _INFO.num_lanes
NW = NC * NS
B_PER_W = BATCH // NW
GRP = B_PER_W // L
QUAD = 4
D_REGS = EMBED_DIM // L


def _body(ids_hbm, table_hbm, bias_hbm, out_hbm,
          idx_v, slabs_v, out_v, bias_v, sem0, sem1, sem2, osem):
    c = lax.axis_index("c")
    s = lax.axis_index("s")
    wid = s * NC + c
    base = wid * B_PER_W

    pltpu.sync_copy(ids_hbm.at[wid], idx_v)
    pltpu.sync_copy(bias_hbm, bias_v)

    bias_regs = [bias_v[pl.ds(k * L, L)] for k in range(D_REGS)]
    lane_iota = lax.iota(jnp.int32, L)
    cvecs = [lane_iota + (k * L) for k in range(D_REGS)]
    sems = (sem0, sem1, sem2)

    def fire_quad(jvec, q, buf, sem):
        for i in range(QUAD):
            col = pl.multiple_of(jvec[q * QUAD + i], LANES)
            pltpu.async_copy(
                table_hbm.at[:, pl.ds(col, LANES)],
                slabs_v.at[buf, i], sem)

    def drain_quad(buf, sem):
        for i in range(QUAD):
            pltpu.make_async_copy(
                table_hbm.at[:, pl.ds(0, LANES)],
                slabs_v.at[buf, i], sem).wait()

    def extract_quad(lvec, q, buf, ob):
        for i in range(QUAD):
            lane_splat = jnp.full((L,), lvec[q * QUAD + i], jnp.int32)
            for k in range(D_REGS):
                row = plsc.load_gather(slabs_v, [
                    jnp.full((L,), buf, jnp.int32),
                    jnp.full((L,), i, jnp.int32),
                    cvecs[k], lane_splat])
                out_v[ob, q * QUAD + i, pl.ds(k * L, L)] = row + bias_regs[k]

    def one_group(g, ob):
        ivec = idx_v[pl.ds(g * L, L)]
        jvec = lax.bitwise_and(ivec, ~(LANES - 1))
        lvec = lax.bitwise_and(ivec, LANES - 1)

        fire_quad(jvec, 0, 0, sems[0])
        fire_quad(jvec, 1, 1, sems[1])
        fire_quad(jvec, 2, 2, sems[2])

        @pl.when(g >= 2)
        def _():
            pltpu.make_async_copy(
                out_v.at[ob], out_hbm.at[pl.ds(base, L)], osem).wait()

        drain_quad(0, sems[0])
        extract_quad(lvec, 0, 0, ob)
        fire_quad(jvec, 3, 0, sems[0])
        drain_quad(1, sems[1])
        extract_quad(lvec, 1, 1, ob)
        drain_quad(2, sems[2])
        extract_quad(lvec, 2, 2, ob)
        drain_quad(0, sems[0])
        extract_quad(lvec, 3, 0, ob)

        pltpu.async_copy(
            out_v.at[ob], out_hbm.at[pl.ds(base + g * L, L)], osem)

    def do_pair(p, carry):
        one_group(p * 2, 0)
        one_group(p * 2 + 1, 1)
        return carry

    lax.fori_loop(0, GRP // 2, do_pair, 0)
    pltpu.make_async_copy(out_v.at[0], out_hbm.at[pl.ds(base, L)], osem).wait()
    pltpu.make_async_copy(out_v.at[1], out_hbm.at[pl.ds(base, L)], osem).wait()


@jax.jit
def _run(ids, table_t, bias):
    mesh = plsc.VectorSubcoreMesh(core_axis_name="c", subcore_axis_name="s")
    f = functools.partial(
        pl.kernel,
        mesh=mesh,
        out_type=jax.ShapeDtypeStruct((BATCH, EMBED_DIM), jnp.float32),
        scratch_types=[
            pltpu.VMEM((B_PER_W,), jnp.int32),
            pltpu.VMEM((3, QUAD, EMBED_DIM, LANES), jnp.float32),
            pltpu.VMEM((2, L, EMBED_DIM), jnp.float32),
            pltpu.VMEM((EMBED_DIM,), jnp.float32),
            pltpu.SemaphoreType.DMA,
            pltpu.SemaphoreType.DMA,
            pltpu.SemaphoreType.DMA,
            pltpu.SemaphoreType.DMA,
        ],
        compiler_params=pltpu.CompilerParams(needs_layout_passes=False),
    )(_body)
    return f(ids, table_t, bias)


def kernel(partition_ids, table, bias):
    ids = partition_ids.astype(jnp.int32).reshape(NW, B_PER_W)
    return _run(ids, table.T, bias)


# speedup vs baseline: 1.3129x; 1.3129x over previous
import functools

import jax
import jax.numpy as jnp
from jax import lax
from jax.experimental import pallas as pl
from jax.experimental.pallas import tpu as pltpu
from jax.experimental.pallas import tpu_sc as plsc

NUM_PARTITIONS = 1000000
EMBED_DIM = 64
BATCH = 16384
LANES = 128

_INFO = plsc.get_sparse_core_info()
NC, NS, L = _INFO.num_cores, _INFO.num_subcores, _INFO.num_lanes
NW = NC * NS
B_PER_W = BATCH // NW
GRP = B_PER_W // L
QUAD = 4
D_REGS = EMBED_DIM // L
IDXCH = 128


def _gather_body(sids_hbm, table_hbm, bias_hbm, rows_hbm,
                 idx_v, slabs_v, out_v, bias_v, sem0, sem1):
    c = lax.axis_index("c")
    s = lax.axis_index("s")
    wid = s * NC + c
    base = wid * B_PER_W

    pltpu.sync_copy(sids_hbm.at[wid], idx_v)
    pltpu.sync_copy(bias_hbm, bias_v)

    bias_regs = [bias_v[pl.ds(k * L, L)] for k in range(D_REGS)]
    lane_iota = lax.iota(jnp.int32, L)
    cvecs = [lane_iota + (k * L) for k in range(D_REGS)]
    sems = (sem0, sem1)

    def quad_meta(jcols, q):
        j = [jcols[q * QUAD + i] for i in range(QUAD)]
        fire = [None, j[1] != j[0], j[2] != j[1], j[3] != j[2]]
        slots = [jnp.int32(0)]
        for i in range(1, QUAD):
            slots.append(slots[i - 1] + fire[i].astype(jnp.int32))
        return j, fire, slots

    def fire_quad(meta, buf, sem):
        j, fire, slots = meta
        col0 = pl.multiple_of(j[0], LANES)
        pltpu.async_copy(
            table_hbm.at[:, pl.ds(col0, LANES)], slabs_v.at[buf, 0], sem)
        for i in range(1, QUAD):
            @pl.when(fire[i])
            def _(i=i):
                col = pl.multiple_of(j[i], LANES)
                pltpu.async_copy(
                    table_hbm.at[:, pl.ds(col, LANES)],
                    slabs_v.at[buf, slots[i]], sem)

    def drain_quad(meta, buf, sem):
        _, fire, _ = meta
        pltpu.make_async_copy(
            table_hbm.at[:, pl.ds(0, LANES)], slabs_v.at[buf, 0], sem).wait()
        for i in range(1, QUAD):
            @pl.when(fire[i])
            def _():
                pltpu.make_async_copy(
                    table_hbm.at[:, pl.ds(0, LANES)],
                    slabs_v.at[buf, 0], sem).wait()

    def extract_quad(meta, lvec, q, buf):
        _, _, slots = meta
        buf_splat = jnp.full((L,), buf, jnp.int32)
        for i in range(QUAD):
            slot_splat = jnp.full((L,), slots[i], jnp.int32)
            lane_splat = jnp.full((L,), lvec[q * QUAD + i], jnp.int32)
            for k in range(D_REGS):
                row = plsc.load_gather(
                    slabs_v, [buf_splat, slot_splat, cvecs[k], lane_splat])
                out_v[q * QUAD + i, pl.ds(k * L, L)] = row + bias_regs[k]

    def do_group(g, carry):
        ivec = idx_v[pl.ds(g * L, L)]
        jvec = lax.bitwise_and(ivec, ~(LANES - 1))
        lvec = lax.bitwise_and(ivec, LANES - 1)
        jcols = [jvec[i] for i in range(L)]
        metas = [quad_meta(jcols, q) for q in range(QUAD)]

        fire_quad(metas[0], 0, sems[0])
        fire_quad(metas[1], 1, sems[1])
        drain_quad(metas[0], 0, sems[0])
        extract_quad(metas[0], lvec, 0, 0)
        fire_quad(metas[2], 0, sems[0])
        drain_quad(metas[1], 1, sems[1])
        extract_quad(metas[1], lvec, 1, 1)
        fire_quad(metas[3], 1, sems[1])
        drain_quad(metas[2], 0, sems[0])
        extract_quad(metas[2], lvec, 2, 0)
        drain_quad(metas[3], 1, sems[1])
        extract_quad(metas[3], lvec, 3, 1)

        pltpu.sync_copy(out_v, rows_hbm.at[pl.ds(base + g * L, L)])
        return carry

    lax.fori_loop(0, GRP, do_group, 0)


def _scatter_body(perm_hbm, rows_hbm, out_hbm, pidx_v, rows_v, sem):
    c = lax.axis_index("c")
    s = lax.axis_index("s")
    wid = s * NC + c
    base = wid * B_PER_W

    pltpu.sync_copy(perm_hbm.at[wid], pidx_v)
    pltpu.sync_copy(rows_hbm.at[pl.ds(base, B_PER_W)], rows_v)

    copies = []
    for q in range(B_PER_W // IDXCH):
        copies.append(pltpu.async_copy(
            rows_v.at[pl.ds(q * IDXCH, IDXCH)],
            out_hbm.at[pidx_v.at[pl.ds(q * IDXCH, IDXCH)]],
            sem))
    for cp in copies:
        cp.wait()


@jax.jit
def _run(ids, table_t, bias):
    perm = jnp.argsort(ids).astype(jnp.int32)
    sids = jnp.take(ids, perm).astype(jnp.int32)

    mesh = plsc.VectorSubcoreMesh(core_axis_name="c", subcore_axis_name="s")
    gather = functools.partial(
        pl.kernel,
        mesh=mesh,
        out_type=jax.ShapeDtypeStruct((BATCH, EMBED_DIM), jnp.float32),
        scratch_types=[
            pltpu.VMEM((B_PER_W,), jnp.int32),
            pltpu.VMEM((2, QUAD, EMBED_DIM, LANES), jnp.float32),
            pltpu.VMEM((L, EMBED_DIM), jnp.float32),
            pltpu.VMEM((EMBED_DIM,), jnp.float32),
            pltpu.SemaphoreType.DMA,
            pltpu.SemaphoreType.DMA,
        ],
        compiler_params=pltpu.CompilerParams(needs_layout_passes=False),
    )(_gather_body)
    rows_sorted = gather(sids.reshape(NW, B_PER_W), table_t, bias)

    scatter = functools.partial(
        pl.kernel,
        mesh=mesh,
        out_type=jax.ShapeDtypeStruct((BATCH, EMBED_DIM), jnp.float32),
        scratch_types=[
            pltpu.VMEM((B_PER_W,), jnp.int32),
            pltpu.VMEM((B_PER_W, EMBED_DIM), jnp.float32),
            pltpu.SemaphoreType.DMA,
        ],
        compiler_params=pltpu.CompilerParams(use_tc_tiling_on_sc=False),
    )(_scatter_body)
    return scatter(perm.reshape(NW, B_PER_W), rows_sorted)


def kernel(partition_ids, table, bias):
    return _run(partition_ids.astype(jnp.int32), table.T, bias)


# speedup vs baseline: 1.5027x; 1.1446x over previous
import functools

import jax
import jax.numpy as jnp
from jax import lax
from jax.experimental import pallas as pl
from jax.experimental.pallas import tpu as pltpu
from jax.experimental.pallas import tpu_sc as plsc

NUM_PARTITIONS = 1000000
EMBED_DIM = 64
BATCH = 16384
LANES = 128
RING = 8

_INFO = plsc.get_sparse_core_info()
NC, NS, L = _INFO.num_cores, _INFO.num_subcores, _INFO.num_lanes
NW = NC * NS
B_PER_W = BATCH // NW
GRP = B_PER_W // L
QUAD = 4
D_REGS = EMBED_DIM // L
IDXCH = 128


def _gather_body(sids_hbm, table_hbm, bias_hbm, rows_hbm,
                 idx_v, slabs_v, out_v, bias_v, sem0, sem1):
    c = lax.axis_index("c")
    s = lax.axis_index("s")
    wid = s * NC + c
    base = wid * B_PER_W

    pltpu.sync_copy(sids_hbm.at[pl.ds(base, B_PER_W)], idx_v)
    pltpu.sync_copy(bias_hbm, bias_v)

    bias_regs = [bias_v[pl.ds(k * L, L)] for k in range(D_REGS)]
    lane_iota = lax.iota(jnp.int32, L)
    cvecs = [lane_iota + (k * L) for k in range(D_REGS)]
    sems = (sem0, sem1)

    def quad_meta(jcols, q, jprev, pos):
        j = [jcols[q * QUAD + i] for i in range(QUAD)]
        fire = [j[0] != jprev]
        for i in range(1, QUAD):
            fire.append(j[i] != j[i - 1])
        slots = []
        cum = pos
        for i in range(QUAD):
            cum = cum + fire[i].astype(jnp.int32)
            slots.append(lax.bitwise_and(cum - 1, RING - 1))
        return (j, fire, slots), j[QUAD - 1], cum

    def fire_quad(meta, sem):
        j, fire, slots = meta
        for i in range(QUAD):
            @pl.when(fire[i])
            def _(i=i):
                col = pl.multiple_of(j[i], LANES)
                pltpu.async_copy(
                    table_hbm.at[:, pl.ds(col, LANES)],
                    slabs_v.at[slots[i]], sem)

    def drain_quad(meta, sem):
        _, fire, _ = meta
        for i in range(QUAD):
            @pl.when(fire[i])
            def _():
                pltpu.make_async_copy(
                    table_hbm.at[:, pl.ds(0, LANES)],
                    slabs_v.at[0], sem).wait()

    def extract_quad(meta, lvec, q):
        _, _, slots = meta
        for i in range(QUAD):
            slot_splat = jnp.full((L,), slots[i], jnp.int32)
            lane_splat = jnp.full((L,), lvec[q * QUAD + i], jnp.int32)
            for k in range(D_REGS):
                row = plsc.load_gather(
                    slabs_v, [slot_splat, cvecs[k], lane_splat])
                out_v[q * QUAD + i, pl.ds(k * L, L)] = row + bias_regs[k]

    def do_group(g, carry):
        jprev, pos = carry
        ivec = idx_v[pl.ds(g * L, L)]
        jvec = lax.bitwise_and(ivec, ~(LANES - 1))
        lvec = lax.bitwise_and(ivec, LANES - 1)
        jcols = [jvec[i] for i in range(L)]

        m0, jprev, pos = quad_meta(jcols, 0, jprev, pos)
        m1, jprev, pos = quad_meta(jcols, 1, jprev, pos)
        m2, jprev, pos = quad_meta(jcols, 2, jprev, pos)
        m3, jprev, pos = quad_meta(jcols, 3, jprev, pos)

        fire_quad(m0, sems[0])
        fire_quad(m1, sems[1])
        drain_quad(m0, sems[0])
        extract_quad(m0, lvec, 0)
        fire_quad(m2, sems[0])
        drain_quad(m1, sems[1])
        extract_quad(m1, lvec, 1)
        fire_quad(m3, sems[1])
        drain_quad(m2, sems[0])
        extract_quad(m2, lvec, 2)
        drain_quad(m3, sems[1])
        extract_quad(m3, lvec, 3)

        pltpu.sync_copy(out_v, rows_hbm.at[pl.ds(base + g * L, L)])
        return (jprev, pos)

    lax.fori_loop(0, GRP, do_group,
                  (jnp.int32(-NUM_PARTITIONS), jnp.int32(0)))


def _scatter_body(perm_hbm, rows_hbm, out_hbm, pidx_v, rows_v, sem):
    c = lax.axis_index("c")
    s = lax.axis_index("s")
    wid = s * NC + c
    base = wid * B_PER_W

    pltpu.sync_copy(perm_hbm.at[pl.ds(base, B_PER_W)], pidx_v)
    pltpu.sync_copy(rows_hbm.at[pl.ds(base, B_PER_W)], rows_v)

    copies = []
    for q in range(B_PER_W // IDXCH):
        copies.append(pltpu.async_copy(
            rows_v.at[pl.ds(q * IDXCH, IDXCH)],
            out_hbm.at[pidx_v.at[pl.ds(q * IDXCH, IDXCH)]],
            sem))
    for cp in copies:
        cp.wait()


@jax.jit
def _run(ids, table_t, bias):
    perm = jnp.argsort(ids).astype(jnp.int32)
    sids = jnp.take(ids, perm).astype(jnp.int32)

    mesh = plsc.VectorSubcoreMesh(core_axis_name="c", subcore_axis_name="s")
    gather = functools.partial(
        pl.kernel,
        mesh=mesh,
        out_type=jax.ShapeDtypeStruct((BATCH, EMBED_DIM), jnp.float32),
        scratch_types=[
            pltpu.VMEM((B_PER_W,), jnp.int32),
            pltpu.VMEM((RING, EMBED_DIM, LANES), jnp.float32),
            pltpu.VMEM((L, EMBED_DIM), jnp.float32),
            pltpu.VMEM((EMBED_DIM,), jnp.float32),
            pltpu.SemaphoreType.DMA,
            pltpu.SemaphoreType.DMA,
        ],
        compiler_params=pltpu.CompilerParams(needs_layout_passes=False),
    )(_gather_body)
    rows_sorted = gather(sids, table_t, bias)

    scatter = functools.partial(
        pl.kernel,
        mesh=mesh,
        out_type=jax.ShapeDtypeStruct((BATCH, EMBED_DIM), jnp.float32),
        scratch_types=[
            pltpu.VMEM((B_PER_W,), jnp.int32),
            pltpu.VMEM((B_PER_W, EMBED_DIM), jnp.float32),
            pltpu.SemaphoreType.DMA,
        ],
        compiler_params=pltpu.CompilerParams(use_tc_tiling_on_sc=False),
    )(_scatter_body)
    return scatter(perm, rows_sorted)


def kernel(partition_ids, table, bias):
    return _run(partition_ids.astype(jnp.int32), table.T, bias)


# speedup vs baseline: 1.5071x; 1.0029x over previous
import functools

import jax
import jax.numpy as jnp
from jax import lax
from jax.experimental import pallas as pl
from jax.experimental.pallas import tpu as pltpu
from jax.experimental.pallas import tpu_sc as plsc

NUM_PARTITIONS = 1000000
EMBED_DIM = 64
BATCH = 16384
LANES = 128
RING = 12

_INFO = plsc.get_sparse_core_info()
NC, NS, L = _INFO.num_cores, _INFO.num_subcores, _INFO.num_lanes
NW = NC * NS
B_PER_W = BATCH // NW
GRP = B_PER_W // L
QUAD = 4
D_REGS = EMBED_DIM // L
IDXCH = 128


def _gather_body(sids_hbm, table_hbm, bias_hbm, rows_hbm,
                 idx_v, slabs_v, out_v, bias_v, sem0, sem1, sem2, osem):
    c = lax.axis_index("c")
    s = lax.axis_index("s")
    wid = s * NC + c
    base = wid * B_PER_W

    pltpu.sync_copy(sids_hbm.at[pl.ds(base, B_PER_W)], idx_v)
    pltpu.sync_copy(bias_hbm, bias_v)

    bias_regs = [bias_v[pl.ds(k * L, L)] for k in range(D_REGS)]
    lane_iota = lax.iota(jnp.int32, L)
    cvecs = [lane_iota + (k * L) for k in range(D_REGS)]
    sems = (sem0, sem1, sem2)

    def quad_meta(jcols, q, jprev, pos):
        j = [jcols[q * QUAD + i] for i in range(QUAD)]
        fire = [j[0] != jprev]
        for i in range(1, QUAD):
            fire.append(j[i] != j[i - 1])
        slots = []
        cum = pos
        for i in range(QUAD):
            cum = cum + fire[i].astype(jnp.int32)
            slots.append(lax.rem(cum - 1, jnp.int32(RING)))
        return (j, fire, slots), j[QUAD - 1], cum

    def fire_quad(meta, sem):
        j, fire, slots = meta
        for i in range(QUAD):
            @pl.when(fire[i])
            def _(i=i):
                col = pl.multiple_of(j[i], LANES)
                pltpu.async_copy(
                    table_hbm.at[:, pl.ds(col, LANES)],
                    slabs_v.at[slots[i]], sem)

    def drain_quad(meta, sem):
        _, fire, _ = meta
        for i in range(QUAD):
            @pl.when(fire[i])
            def _():
                pltpu.make_async_copy(
                    table_hbm.at[:, pl.ds(0, LANES)],
                    slabs_v.at[0], sem).wait()

    def extract_quad(meta, lvec, q, ob):
        _, _, slots = meta
        for i in range(QUAD):
            slot_splat = jnp.full((L,), slots[i], jnp.int32)
            lane_splat = jnp.full((L,), lvec[q * QUAD + i], jnp.int32)
            for k in range(D_REGS):
                row = plsc.load_gather(
                    slabs_v, [slot_splat, cvecs[k], lane_splat])
                out_v[ob, q * QUAD + i, pl.ds(k * L, L)] = row + bias_regs[k]

    def one_group(g, ob, carry):
        jprev, pos = carry
        ivec = idx_v[pl.ds(g * L, L)]
        jvec = lax.bitwise_and(ivec, ~(LANES - 1))
        lvec = lax.bitwise_and(ivec, LANES - 1)
        jcols = [jvec[i] for i in range(L)]

        m0, jprev, pos = quad_meta(jcols, 0, jprev, pos)
        m1, jprev, pos = quad_meta(jcols, 1, jprev, pos)
        m2, jprev, pos = quad_meta(jcols, 2, jprev, pos)
        m3, jprev, pos = quad_meta(jcols, 3, jprev, pos)

        fire_quad(m0, sems[0])
        fire_quad(m1, sems[1])
        fire_quad(m2, sems[2])

        @pl.when(g >= 2)
        def _():
            pltpu.make_async_copy(
                out_v.at[ob], rows_hbm.at[pl.ds(base, L)], osem).wait()

        drain_quad(m0, sems[0])
        extract_quad(m0, lvec, 0, ob)
        fire_quad(m3, sems[0])
        drain_quad(m1, sems[1])
        extract_quad(m1, lvec, 1, ob)
        drain_quad(m2, sems[2])
        extract_quad(m2, lvec, 2, ob)
        drain_quad(m3, sems[0])
        extract_quad(m3, lvec, 3, ob)

        pltpu.async_copy(
            out_v.at[ob], rows_hbm.at[pl.ds(base + g * L, L)], osem)
        return (jprev, pos)

    def do_pair(p, carry):
        carry = one_group(p * 2, 0, carry)
        carry = one_group(p * 2 + 1, 1, carry)
        return carry

    lax.fori_loop(0, GRP // 2, do_pair,
                  (jnp.int32(-NUM_PARTITIONS), jnp.int32(0)))
    pltpu.make_async_copy(out_v.at[0], rows_hbm.at[pl.ds(base, L)],
                          osem).wait()
    pltpu.make_async_copy(out_v.at[1], rows_hbm.at[pl.ds(base, L)],
                          osem).wait()


def _scatter_body(perm_hbm, rows_hbm, out_hbm, pidx_v, rows_v, sem):
    c = lax.axis_index("c")
    s = lax.axis_index("s")
    wid = s * NC + c
    base = wid * B_PER_W

    pltpu.sync_copy(perm_hbm.at[pl.ds(base, B_PER_W)], pidx_v)
    pltpu.sync_copy(rows_hbm.at[pl.ds(base, B_PER_W)], rows_v)

    copies = []
    for q in range(B_PER_W // IDXCH):
        copies.append(pltpu.async_copy(
            rows_v.at[pl.ds(q * IDXCH, IDXCH)],
            out_hbm.at[pidx_v.at[pl.ds(q * IDXCH, IDXCH)]],
            sem))
    for cp in copies:
        cp.wait()


@jax.jit
def _run(ids, table_t, bias):
    perm = jnp.argsort(ids).astype(jnp.int32)
    sids = jnp.take(ids, perm).astype(jnp.int32)

    mesh = plsc.VectorSubcoreMesh(core_axis_name="c", subcore_axis_name="s")
    gather = functools.partial(
        pl.kernel,
        mesh=mesh,
        out_type=jax.ShapeDtypeStruct((BATCH, EMBED_DIM), jnp.float32),
        scratch_types=[
            pltpu.VMEM((B_PER_W,), jnp.int32),
            pltpu.VMEM((RING, EMBED_DIM, LANES), jnp.float32),
            pltpu.VMEM((2, L, EMBED_DIM), jnp.float32),
            pltpu.VMEM((EMBED_DIM,), jnp.float32),
            pltpu.SemaphoreType.DMA,
            pltpu.SemaphoreType.DMA,
            pltpu.SemaphoreType.DMA,
            pltpu.SemaphoreType.DMA,
        ],
        compiler_params=pltpu.CompilerParams(needs_layout_passes=False),
    )(_gather_body)
    rows_sorted = gather(sids, table_t, bias)

    scatter = functools.partial(
        pl.kernel,
        mesh=mesh,
        out_type=jax.ShapeDtypeStruct((BATCH, EMBED_DIM), jnp.float32),
        scratch_types=[
            pltpu.VMEM((B_PER_W,), jnp.int32),
            pltpu.VMEM((B_PER_W, EMBED_DIM), jnp.float32),
            pltpu.SemaphoreType.DMA,
        ],
        compiler_params=pltpu.CompilerParams(use_tc_tiling_on_sc=False),
    )(_scatter_body)
    return scatter(perm, rows_sorted)


def kernel(partition_ids, table, bias):
    return _run(partition_ids.astype(jnp.int32), table.T, bias)


# speedup vs baseline: 1.5691x; 1.0411x over previous
import functools

import jax
import jax.numpy as jnp
from jax import lax
from jax.experimental import pallas as pl
from jax.experimental.pallas import tpu as pltpu
from jax.experimental.pallas import tpu_sc as plsc

NUM_PARTITIONS = 1000000
EMBED_DIM = 64
BATCH = 16384
LANES = 128
RING = 12

_INFO = plsc.get_sparse_core_info()
NC, NS, L = _INFO.num_cores, _INFO.num_subcores, _INFO.num_lanes
NW = NC * NS
B_PER_W = BATCH // NW
GRP = B_PER_W // L
QUAD = 4
D_REGS = EMBED_DIM // L
IDXCH = 128


def _gather_body(sids_hbm, table_hbm, bias_hbm, rows_hbm,
                 idx_v, slabs_v, out_v, bias_v, sem0, sem1, sem2, osem):
    c = lax.axis_index("c")
    s = lax.axis_index("s")
    wid = s * NC + c
    base = wid * B_PER_W

    pltpu.sync_copy(sids_hbm.at[pl.ds(base, B_PER_W)], idx_v)
    pltpu.sync_copy(bias_hbm, bias_v)

    bias_regs = [bias_v[pl.ds(k * L, L)] for k in range(D_REGS)]
    lane_iota = lax.iota(jnp.int32, L)
    cvecs = [lane_iota + (k * L) for k in range(D_REGS)]
    sems = (sem0, sem1, sem2)

    def quad_meta(jcols, q, jprev, pos):
        j = [jcols[q * QUAD + i] for i in range(QUAD)]
        fire = [j[0] != jprev]
        for i in range(1, QUAD):
            fire.append(j[i] != j[i - 1])
        slots = []
        cum = pos
        for i in range(QUAD):
            cum = cum + fire[i].astype(jnp.int32)
            slots.append(lax.rem(cum - 1, jnp.int32(RING)))
        return (j, fire, slots), j[QUAD - 1], cum

    def fire_quad(meta, sem):
        j, fire, slots = meta
        for i in range(QUAD):
            @pl.when(fire[i])
            def _(i=i):
                col = pl.multiple_of(j[i], LANES)
                pltpu.async_copy(
                    table_hbm.at[:, pl.ds(col, LANES)],
                    slabs_v.at[slots[i]], sem)

    def drain_quad(meta, sem):
        _, fire, _ = meta
        for i in range(QUAD):
            @pl.when(fire[i])
            def _():
                pltpu.make_async_copy(
                    table_hbm.at[:, pl.ds(0, LANES)],
                    slabs_v.at[0], sem).wait()

    def extract_quad(meta, lvec, q, ob):
        _, _, slots = meta
        for i in range(QUAD):
            slot_splat = jnp.full((L,), slots[i], jnp.int32)
            lane_splat = jnp.full((L,), lvec[q * QUAD + i], jnp.int32)
            for k in range(D_REGS):
                row = plsc.load_gather(
                    slabs_v, [slot_splat, cvecs[k], lane_splat])
                out_v[ob, q * QUAD + i, pl.ds(k * L, L)] = row + bias_regs[k]

    def one_group(g, ob, carry):
        jprev, pos = carry
        ivec = idx_v[pl.ds(g * L, L)]
        jvec = lax.bitwise_and(ivec, ~(LANES - 1))
        lvec = lax.bitwise_and(ivec, LANES - 1)
        jcols = [jvec[i] for i in range(L)]

        m0, jprev, pos = quad_meta(jcols, 0, jprev, pos)
        m1, jprev, pos = quad_meta(jcols, 1, jprev, pos)
        m2, jprev, pos = quad_meta(jcols, 2, jprev, pos)
        m3, jprev, pos = quad_meta(jcols, 3, jprev, pos)

        fire_quad(m0, sems[0])
        fire_quad(m1, sems[1])
        fire_quad(m2, sems[2])

        @pl.when(g >= 2)
        def _():
            pltpu.make_async_copy(
                out_v.at[ob], rows_hbm.at[pl.ds(base, L)], osem).wait()

        drain_quad(m0, sems[0])
        extract_quad(m0, lvec, 0, ob)
        fire_quad(m3, sems[0])
        drain_quad(m1, sems[1])
        extract_quad(m1, lvec, 1, ob)
        drain_quad(m2, sems[2])
        extract_quad(m2, lvec, 2, ob)
        drain_quad(m3, sems[0])
        extract_quad(m3, lvec, 3, ob)

        pltpu.async_copy(
            out_v.at[ob], rows_hbm.at[pl.ds(base + g * L, L)], osem)
        return (jprev, pos)

    def do_pair(p, carry):
        carry = one_group(p * 2, 0, carry)
        carry = one_group(p * 2 + 1, 1, carry)
        return carry

    lax.fori_loop(0, GRP // 2, do_pair,
                  (jnp.int32(-NUM_PARTITIONS), jnp.int32(0)))
    pltpu.make_async_copy(out_v.at[0], rows_hbm.at[pl.ds(base, L)],
                          osem).wait()
    pltpu.make_async_copy(out_v.at[1], rows_hbm.at[pl.ds(base, L)],
                          osem).wait()


def _scatter_body(perm_hbm, rows_hbm, out_hbm, pidx_v, rows_v, sem):
    c = lax.axis_index("c")
    s = lax.axis_index("s")
    wid = s * NC + c
    base = wid * B_PER_W

    pltpu.sync_copy(perm_hbm.at[pl.ds(base, B_PER_W)], pidx_v)
    pltpu.sync_copy(rows_hbm.at[pl.ds(base, B_PER_W)], rows_v)

    copies = []
    for q in range(B_PER_W // IDXCH):
        copies.append(pltpu.async_copy(
            rows_v.at[pl.ds(q * IDXCH, IDXCH)],
            out_hbm.at[pidx_v.at[pl.ds(q * IDXCH, IDXCH)]],
            sem))
    for cp in copies:
        cp.wait()


@jax.jit
def _run(ids, table_t, bias):
    sids, perm = lax.sort_key_val(
        ids, lax.iota(jnp.int32, BATCH), dimension=0)

    mesh = plsc.VectorSubcoreMesh(core_axis_name="c", subcore_axis_name="s")
    gather = functools.partial(
        pl.kernel,
        mesh=mesh,
        out_type=jax.ShapeDtypeStruct((BATCH, EMBED_DIM), jnp.float32),
        scratch_types=[
            pltpu.VMEM((B_PER_W,), jnp.int32),
            pltpu.VMEM((RING, EMBED_DIM, LANES), jnp.float32),
            pltpu.VMEM((2, L, EMBED_DIM), jnp.float32),
            pltpu.VMEM((EMBED_DIM,), jnp.float32),
            pltpu.SemaphoreType.DMA,
            pltpu.SemaphoreType.DMA,
            pltpu.SemaphoreType.DMA,
            pltpu.SemaphoreType.DMA,
        ],
        compiler_params=pltpu.CompilerParams(needs_layout_passes=False),
    )(_gather_body)
    rows_sorted = gather(sids, table_t, bias)

    scatter = functools.partial(
        pl.kernel,
        mesh=mesh,
        out_type=jax.ShapeDtypeStruct((BATCH, EMBED_DIM), jnp.float32),
        scratch_types=[
            pltpu.VMEM((B_PER_W,), jnp.int32),
            pltpu.VMEM((B_PER_W, EMBED_DIM), jnp.float32),
            pltpu.SemaphoreType.DMA,
        ],
        compiler_params=pltpu.CompilerParams(use_tc_tiling_on_sc=False),
    )(_scatter_body)
    return scatter(perm, rows_sorted)


def kernel(partition_ids, table, bias):
    return _run(partition_ids.astype(jnp.int32), table.T, bias)


# speedup vs baseline: 1.6179x; 1.0311x over previous
import functools

import jax
import jax.numpy as jnp
from jax import lax
from jax.experimental import pallas as pl
from jax.experimental.pallas import tpu as pltpu
from jax.experimental.pallas import tpu_sc as plsc

NUM_PARTITIONS = 1000000
EMBED_DIM = 64
BATCH = 16384
LANES = 128
RING = 12

_INFO = plsc.get_sparse_core_info()
NC, NS, L = _INFO.num_cores, _INFO.num_subcores, _INFO.num_lanes
NW = NC * NS
B_PER_W = BATCH // NW
GRP = B_PER_W // L
QUAD = 4
D_REGS = EMBED_DIM // L
IDXCH = 128


def _gather_body(sids_hbm, table_hbm, bias_hbm, rows_hbm,
                 idx_v, slabs_v, out_v, bias_v, sem0, sem1, sem2, osem):
    c = lax.axis_index("c")
    s = lax.axis_index("s")
    wid = s * NC + c
    base = wid * B_PER_W

    pltpu.sync_copy(sids_hbm.at[pl.ds(base, B_PER_W)], idx_v)
    pltpu.sync_copy(bias_hbm, bias_v)

    bias_regs = [bias_v[pl.ds(k * L, L)] for k in range(D_REGS)]
    lane_iota = lax.iota(jnp.int32, L)
    cvecs = [lane_iota + (k * L) for k in range(D_REGS)]
    sems = (sem0, sem1, sem2)

    def quad_meta(jcols, q, jprev, pos):
        j = [jcols[q * QUAD + i] for i in range(QUAD)]
        fire = [j[0] != jprev]
        for i in range(1, QUAD):
            fire.append(j[i] != j[i - 1])
        slots = []
        cum = pos
        for i in range(QUAD):
            cum = cum + fire[i].astype(jnp.int32)
            slots.append(lax.rem(cum - 1, jnp.int32(RING)))
        return (j, fire, slots), j[QUAD - 1], cum

    def fire_quad(meta, sem):
        j, fire, slots = meta
        for i in range(QUAD):
            @pl.when(fire[i])
            def _(i=i):
                col = pl.multiple_of(j[i], LANES)
                pltpu.async_copy(
                    table_hbm.at[:, pl.ds(col, LANES)],
                    slabs_v.at[slots[i]], sem)

    def drain_quad(meta, sem):
        _, fire, _ = meta
        for i in range(QUAD):
            @pl.when(fire[i])
            def _():
                pltpu.make_async_copy(
                    table_hbm.at[:, pl.ds(0, LANES)],
                    slabs_v.at[0], sem).wait()

    def extract_quad(meta, lvec, q, ob):
        _, _, slots = meta
        for i in range(QUAD):
            slot_splat = jnp.full((L,), slots[i], jnp.int32)
            lane_splat = jnp.full((L,), lvec[q * QUAD + i], jnp.int32)
            for k in range(D_REGS):
                row = plsc.load_gather(
                    slabs_v, [slot_splat, cvecs[k], lane_splat])
                out_v[ob, pl.ds((q * QUAD + i) * EMBED_DIM + k * L, L)] = (
                    row + bias_regs[k])

    def one_group(g, ob, carry):
        jprev, pos = carry
        ivec = idx_v[pl.ds(g * L, L)]
        jvec = lax.bitwise_and(ivec, ~(LANES - 1))
        lvec = lax.bitwise_and(ivec, LANES - 1)
        jcols = [jvec[i] for i in range(L)]

        m0, jprev, pos = quad_meta(jcols, 0, jprev, pos)
        m1, jprev, pos = quad_meta(jcols, 1, jprev, pos)
        m2, jprev, pos = quad_meta(jcols, 2, jprev, pos)
        m3, jprev, pos = quad_meta(jcols, 3, jprev, pos)

        fire_quad(m0, sems[0])
        fire_quad(m1, sems[1])
        fire_quad(m2, sems[2])

        @pl.when(g >= 2)
        def _():
            pltpu.make_async_copy(
                out_v.at[ob], rows_hbm.at[pl.ds(0, L * EMBED_DIM)],
                osem).wait()

        drain_quad(m0, sems[0])
        extract_quad(m0, lvec, 0, ob)
        fire_quad(m3, sems[0])
        drain_quad(m1, sems[1])
        extract_quad(m1, lvec, 1, ob)
        drain_quad(m2, sems[2])
        extract_quad(m2, lvec, 2, ob)
        drain_quad(m3, sems[0])
        extract_quad(m3, lvec, 3, ob)

        pltpu.async_copy(
            out_v.at[ob],
            rows_hbm.at[pl.ds((base + g * L) * EMBED_DIM, L * EMBED_DIM)],
            osem)
        return (jprev, pos)

    def do_pair(p, carry):
        carry = one_group(p * 2, 0, carry)
        carry = one_group(p * 2 + 1, 1, carry)
        return carry

    lax.fori_loop(0, GRP // 2, do_pair,
                  (jnp.int32(-NUM_PARTITIONS), jnp.int32(0)))
    pltpu.make_async_copy(out_v.at[0],
                          rows_hbm.at[pl.ds(0, L * EMBED_DIM)], osem).wait()
    pltpu.make_async_copy(out_v.at[1],
                          rows_hbm.at[pl.ds(0, L * EMBED_DIM)], osem).wait()


def _scatter_body(perm_hbm, rows_hbm, out_hbm, pidx_v, rows_v, sem):
    c = lax.axis_index("c")
    s = lax.axis_index("s")
    wid = s * NC + c
    base = wid * B_PER_W

    pltpu.sync_copy(perm_hbm.at[pl.ds(base, B_PER_W)], pidx_v)
    pltpu.sync_copy(rows_hbm.at[pl.ds(base, B_PER_W)], rows_v)

    copies = []
    for q in range(B_PER_W // IDXCH):
        copies.append(pltpu.async_copy(
            rows_v.at[pl.ds(q * IDXCH, IDXCH)],
            out_hbm.at[pidx_v.at[pl.ds(q * IDXCH, IDXCH)]],
            sem))
    for cp in copies:
        cp.wait()


@jax.jit
def _run(ids, table_t, bias):
    sids, perm = lax.sort_key_val(
        ids, lax.iota(jnp.int32, BATCH), dimension=0)

    mesh = plsc.VectorSubcoreMesh(core_axis_name="c", subcore_axis_name="s")
    gather = functools.partial(
        pl.kernel,
        mesh=mesh,
        out_type=jax.ShapeDtypeStruct((BATCH * EMBED_DIM,), jnp.float32),
        scratch_types=[
            pltpu.VMEM((B_PER_W,), jnp.int32),
            pltpu.VMEM((RING, EMBED_DIM, LANES), jnp.float32),
            pltpu.VMEM((2, L * EMBED_DIM), jnp.float32),
            pltpu.VMEM((EMBED_DIM,), jnp.float32),
            pltpu.SemaphoreType.DMA,
            pltpu.SemaphoreType.DMA,
            pltpu.SemaphoreType.DMA,
            pltpu.SemaphoreType.DMA,
        ],
        compiler_params=pltpu.CompilerParams(needs_layout_passes=False),
    )(_gather_body)
    rows_sorted = gather(sids, table_t, bias).reshape(BATCH, EMBED_DIM)

    scatter = functools.partial(
        pl.kernel,
        mesh=mesh,
        out_type=jax.ShapeDtypeStruct((BATCH, EMBED_DIM), jnp.float32),
        scratch_types=[
            pltpu.VMEM((B_PER_W,), jnp.int32),
            pltpu.VMEM((B_PER_W, EMBED_DIM), jnp.float32),
            pltpu.SemaphoreType.DMA,
        ],
        compiler_params=pltpu.CompilerParams(use_tc_tiling_on_sc=False),
    )(_scatter_body)
    return scatter(perm, rows_sorted)


def kernel(partition_ids, table, bias):
    return _run(partition_ids.astype(jnp.int32), table.T, bias)
